# baseline (speedup 1.0000x reference)
import functools

import jax
import jax.numpy as jnp
from jax import lax
from jax.experimental import pallas as pl
from jax.experimental.pallas import tpu as pltpu
from jax.experimental.pallas import tpu_sc as plsc

L = 16
NW = 32
FG_BG_THRESH = 0.5


def _make_sc_call(n, g):
    chunk = (n // NW) // L * L
    last = n - (NW - 1) * chunk
    assert last > 0 and last % L == 0 and chunk % L == 0 and chunk % 8 == 0
    nblk = chunk // L
    nblk_last = last // L
    cbuf = max(chunk, last)
    mesh = plsc.VectorSubcoreMesh(core_axis_name="c", subcore_axis_name="s")

    steps = []
    s = g // 2
    while s >= 1:
        steps.append(s)
        s //= 2

    @functools.partial(
        pl.kernel,
        mesh=mesh,
        compiler_params=pltpu.CompilerParams(needs_layout_passes=False),
        out_type=[
            jax.ShapeDtypeStruct((n,), jnp.int32),
            jax.ShapeDtypeStruct((n,), jnp.float32),
            jax.ShapeDtypeStruct((n,), jnp.int32),
        ],
        scratch_types=[
            pltpu.VMEM((cbuf,), jnp.float32),
            pltpu.VMEM((cbuf,), jnp.float32),
            pltpu.VMEM((cbuf,), jnp.float32),
            pltpu.VMEM((cbuf,), jnp.float32),
            pltpu.VMEM((g,), jnp.float32),
            pltpu.VMEM((g,), jnp.float32),
            pltpu.VMEM((g,), jnp.float32),
            pltpu.VMEM((g,), jnp.float32),
            pltpu.VMEM((g + L,), jnp.float32),
            pltpu.VMEM((g + L,), jnp.float32),
            pltpu.VMEM((g + L,), jnp.float32),
            pltpu.VMEM((g + L,), jnp.float32),
            pltpu.VMEM((g,), jnp.int32),
            pltpu.VMEM((g,), jnp.int32),
            pltpu.VMEM((cbuf,), jnp.int32),
            pltpu.VMEM((cbuf,), jnp.float32),
            pltpu.VMEM((cbuf,), jnp.int32),
        ],
    )
    def sc_call(px0_hbm, py0_hbm, px1_hbm, py1_hbm,
                rx0_hbm, ry0_hbm, rx1_hbm, ry1_hbm, ord_hbm, gtl_hbm,
                lab_hbm, mv_hbm, idx_hbm,
                px0_v, py0_v, px1_v, py1_v,
                rx0_v, ry0_v, rx1_v, ry1_v,
                gx0_v, gy0_v, gx1_v, gy1_v, orig_v, gtl_v,
                lab_v, mv_v, idx_v):
        cid = lax.axis_index("c")
        sid = lax.axis_index("s")
        wid = sid * 2 + cid
        base = wid * chunk
        is_last = wid == NW - 1

        pltpu.sync_copy(rx0_hbm, rx0_v)
        pltpu.sync_copy(ry0_hbm, ry0_v)
        pltpu.sync_copy(rx1_hbm, rx1_v)
        pltpu.sync_copy(ry1_hbm, ry1_v)
        pltpu.sync_copy(ord_hbm, orig_v)
        pltpu.sync_copy(gtl_hbm, gtl_v)

        @pl.when(jnp.logical_not(is_last))
        def _():
            for hbm, v in ((px0_hbm, px0_v), (py0_hbm, py0_v),
                           (px1_hbm, px1_v), (py1_hbm, py1_v)):
                pltpu.sync_copy(hbm.at[pl.ds(base, chunk)],
                                v.at[pl.ds(0, chunk)])

        @pl.when(is_last)
        def _():
            for hbm, v in ((px0_hbm, px0_v), (py0_hbm, py0_v),
                           (px1_hbm, px1_v), (py1_hbm, py1_v)):
                pltpu.sync_copy(hbm.at[pl.ds(base, last)],
                                v.at[pl.ds(0, last)])

        big = jnp.full((L,), 1.0e30, jnp.float32)
        gx0_v[pl.ds(g, L)] = big
        gy0_v[pl.ds(g, L)] = big
        gx1_v[pl.ds(g, L)] = big
        gy1_v[pl.ds(g, L)] = big

        mw = jnp.zeros((L,), jnp.float32)
        for i in range(g // L):
            ordv = orig_v[pl.ds(i * L, L)]
            x0v = plsc.load_gather(rx0_v, [ordv])
            x1v = plsc.load_gather(rx1_v, [ordv])
            gx0_v[pl.ds(i * L, L)] = x0v
            gy0_v[pl.ds(i * L, L)] = plsc.load_gather(ry0_v, [ordv])
            gx1_v[pl.ds(i * L, L)] = x1v
            gy1_v[pl.ds(i * L, L)] = plsc.load_gather(ry1_v, [ordv])
            mw = jnp.maximum(mw, x1v - x0v)
        maxw = jnp.max(mw)

        zi = jnp.zeros((L,), jnp.int32)
        zf = jnp.zeros((L,), jnp.float32)
        c0 = zi

        def block(j, _):
            off = j * L
            px0 = px0_v[pl.ds(off, L)]
            py0 = py0_v[pl.ds(off, L)]
            px1 = px1_v[pl.ds(off, L)]
            py1 = py1_v[pl.ds(off, L)]
            parea = (px1 - px0) * (py1 - py0)

            tlo = px0 - maxw
            lo = zi
            hi = zi
            for s in steps:
                vlo = plsc.load_gather(gx0_v, [lo + (s - 1)])
                lo = jnp.where(vlo < tlo, lo + s, lo)
                vhi = plsc.load_gather(gx0_v, [hi + (s - 1)])
                hi = jnp.where(vhi <= px1, hi + s, hi)
            vlo = plsc.load_gather(gx0_v, [lo])
            lo = jnp.where(vlo < tlo, lo + 1, lo)
            vhi = plsc.load_gather(gx0_v, [hi])
            hi = jnp.where(vhi <= px1, hi + 1, hi)

            trip = jnp.max(hi - lo)
            send = jnp.full((L,), g, jnp.int32)

            def citer(k, carry):
                best, bidx = carry
                safe = jnp.minimum(lo + k, send)
                gx0 = plsc.load_gather(gx0_v, [safe])
                gy0 = plsc.load_gather(gy0_v, [safe])
                gx1 = plsc.load_gather(gx1_v, [safe])
                gy1 = plsc.load_gather(gy1_v, [safe])
                ga = (gx1 - gx0) * (gy1 - gy0)
                w = jnp.maximum(
                    jnp.minimum(px1, gx1) - jnp.maximum(px0, gx0), 0.0)
                h = jnp.maximum(
                    jnp.minimum(py1, gy1) - jnp.maximum(py0, gy0), 0.0)
                inter = w * h
                union = (parea + ga) - inter
                iou = inter / union
                upd = iou > best
                best = jnp.where(upd, iou, best)
                bidx = jnp.where(upd, safe, bidx)
                return best, bidx

            best, bidx = plsc.parallel_loop(
                0, trip, 1, unroll=4, carry=(zf, zi))(citer)

            below = best < FG_BG_THRESH
            orig = plsc.load_gather(orig_v, [bidx])
            labs = plsc.load_gather(gtl_v, [orig])
            cidx = jnp.where(below, zi, orig)
            labs = jnp.where(below, zi, labs)
            mv_v[pl.ds(off, L)] = best
            idx_v[pl.ds(off, L)] = cidx
            lab_v[pl.ds(off, L)] = labs
            return 0

        lax.fori_loop(0, jnp.where(is_last, nblk_last, nblk), block, 0)

        @pl.when(jnp.logical_not(is_last))
        def _():
            pltpu.sync_copy(lab_v.at[pl.ds(0, chunk)],
                            lab_hbm.at[pl.ds(base, chunk)])
            pltpu.sync_copy(mv_v.at[pl.ds(0, chunk)],
                            mv_hbm.at[pl.ds(base, chunk)])
            pltpu.sync_copy(idx_v.at[pl.ds(0, chunk)],
                            idx_hbm.at[pl.ds(base, chunk)])

        @pl.when(is_last)
        def _():
            pltpu.sync_copy(lab_v.at[pl.ds(0, last)],
                            lab_hbm.at[pl.ds(base, last)])
            pltpu.sync_copy(mv_v.at[pl.ds(0, last)],
                            mv_hbm.at[pl.ds(base, last)])
            pltpu.sync_copy(idx_v.at[pl.ds(0, last)],
                            idx_hbm.at[pl.ds(base, last)])

    return sc_call


def _make_tc_call(m, g):
    rows = m // 128
    grid = (rows // 8,)

    def body(px0_ref, py0_ref, px1_ref, py1_ref,
             gx0_ref, gy0_ref, gx1_ref, gy1_ref,
             mv_ref, idx_ref):
        px0 = px0_ref[...]
        py0 = py0_ref[...]
        px1 = px1_ref[...]
        py1 = py1_ref[...]
        parea = (px1 - px0) * (py1 - py0)
        zf = jnp.zeros((8, 128), jnp.float32)
        zi = jnp.zeros((8, 128), jnp.int32)

        def giter(gi, carry):
            best, bidx = carry
            gx0 = gx0_ref[gi]
            gy0 = gy0_ref[gi]
            gx1 = gx1_ref[gi]
            gy1 = gy1_ref[gi]
            garea = (gx1 - gx0) * (gy1 - gy0)
            w = jnp.maximum(
                jnp.minimum(px1, gx1) - jnp.maximum(px0, gx0), 0.0)
            h = jnp.maximum(
                jnp.minimum(py1, gy1) - jnp.maximum(py0, gy0), 0.0)
            inter = w * h
            union = (parea + garea) - inter
            iou = inter / union
            upd = iou > best
            best = jnp.where(upd, iou, best)
            bidx = jnp.where(upd, jnp.full((8, 128), gi, jnp.int32), bidx)
            return best, bidx

        best, bidx = lax.fori_loop(0, g, giter, (zf, zi))
        mv_ref[...] = best
        idx_ref[...] = jnp.where(best < FG_BG_THRESH, 0, bidx)

    blk = pl.BlockSpec((8, 128), lambda i: (i, 0))
    smem = pl.BlockSpec(memory_space=pltpu.SMEM)
    return pl.pallas_call(
        body,
        grid=grid,
        in_specs=[blk] * 4 + [smem] * 4,
        out_specs=[blk, blk],
        out_shape=[
            jax.ShapeDtypeStruct((rows, 128), jnp.float32),
            jax.ShapeDtypeStruct((rows, 128), jnp.int32),
        ],
    )


def kernel(proposals, gt_boxes, gt_labels):
    n = proposals.shape[0]
    g = gt_boxes.shape[0]
    tc_m = min((n * 21 // 40) // 1024 * 1024, n - NW * L)
    s = n - tc_m
    order = jnp.argsort(gt_boxes[:, 0]).astype(jnp.int32)
    px0, py0 = proposals[:, 0], proposals[:, 1]
    px1, py1 = proposals[:, 2], proposals[:, 3]
    gx0, gy0 = gt_boxes[:, 0], gt_boxes[:, 1]
    gx1, gy1 = gt_boxes[:, 2], gt_boxes[:, 3]
    lab_sc, mv_sc, idx_sc = _make_sc_call(s, g)(
        px0[:s], py0[:s], px1[:s], py1[:s],
        gx0, gy0, gx1, gy1, order, gt_labels)
    rows = tc_m // 128
    mv_tc, idx_tc = _make_tc_call(tc_m, g)(
        px0[s:].reshape(rows, 128), py0[s:].reshape(rows, 128),
        px1[s:].reshape(rows, 128), py1[s:].reshape(rows, 128),
        gx0, gy0, gx1, gy1)
    mv_tc = mv_tc.reshape(-1)
    idx_tc = idx_tc.reshape(-1)
    lab_tc = jnp.where(mv_tc < FG_BG_THRESH, 0,
                       jnp.take(gt_labels, idx_tc))
    lab = jnp.concatenate([lab_sc, lab_tc])
    mv = jnp.concatenate([mv_sc, mv_tc])
    idx = jnp.concatenate([idx_sc, idx_tc])
    return lab, mv, idx


# speedup vs baseline: 3.4516x; 3.4516x over previous
import functools

import jax
import jax.numpy as jnp
from jax import lax
from jax.experimental import pallas as pl
from jax.experimental.pallas import tpu as pltpu
from jax.experimental.pallas import tpu_sc as plsc

L = 16
NW = 32
FG_BG_THRESH = 0.5


def _make_sc_call(n, g):
    chunk = (n // NW) // L * L
    last = n - (NW - 1) * chunk
    assert last > 0 and last % L == 0 and chunk % L == 0 and chunk % 8 == 0
    nblk = chunk // L
    nblk_last = last // L
    cbuf = max(chunk, last)
    mesh = plsc.VectorSubcoreMesh(core_axis_name="c", subcore_axis_name="s")

    steps = []
    s = g // 2
    while s >= 1:
        steps.append(s)
        s //= 2

    @functools.partial(
        pl.kernel,
        mesh=mesh,
        compiler_params=pltpu.CompilerParams(needs_layout_passes=False),
        out_type=[
            jax.ShapeDtypeStruct((n,), jnp.int32),
            jax.ShapeDtypeStruct((n,), jnp.float32),
            jax.ShapeDtypeStruct((n,), jnp.int32),
        ],
        scratch_types=[
            pltpu.VMEM((cbuf,), jnp.float32),
            pltpu.VMEM((cbuf,), jnp.float32),
            pltpu.VMEM((cbuf,), jnp.float32),
            pltpu.VMEM((cbuf,), jnp.float32),
            pltpu.VMEM((g,), jnp.float32),
            pltpu.VMEM((g,), jnp.float32),
            pltpu.VMEM((g,), jnp.float32),
            pltpu.VMEM((g,), jnp.float32),
            pltpu.VMEM((g + L,), jnp.float32),
            pltpu.VMEM((g + L,), jnp.float32),
            pltpu.VMEM((g + L,), jnp.float32),
            pltpu.VMEM((g + L,), jnp.float32),
            pltpu.VMEM((g,), jnp.int32),
            pltpu.VMEM((g,), jnp.int32),
            pltpu.VMEM((cbuf,), jnp.int32),
            pltpu.VMEM((cbuf,), jnp.float32),
            pltpu.VMEM((cbuf,), jnp.int32),
        ],
    )
    def sc_call(px0_hbm, py0_hbm, px1_hbm, py1_hbm,
                rx0_hbm, ry0_hbm, rx1_hbm, ry1_hbm, ord_hbm, gtl_hbm,
                lab_hbm, mv_hbm, idx_hbm,
                px0_v, py0_v, px1_v, py1_v,
                rx0_v, ry0_v, rx1_v, ry1_v,
                gx0_v, gy0_v, gx1_v, gy1_v, orig_v, gtl_v,
                lab_v, mv_v, idx_v):
        cid = lax.axis_index("c")
        sid = lax.axis_index("s")
        wid = sid * 2 + cid
        base = wid * chunk
        is_last = wid == NW - 1

        pltpu.sync_copy(rx0_hbm, rx0_v)
        pltpu.sync_copy(ry0_hbm, ry0_v)
        pltpu.sync_copy(rx1_hbm, rx1_v)
        pltpu.sync_copy(ry1_hbm, ry1_v)
        pltpu.sync_copy(ord_hbm, orig_v)
        pltpu.sync_copy(gtl_hbm, gtl_v)

        @pl.when(jnp.logical_not(is_last))
        def _():
            for hbm, v in ((px0_hbm, px0_v), (py0_hbm, py0_v),
                           (px1_hbm, px1_v), (py1_hbm, py1_v)):
                pltpu.sync_copy(hbm.at[pl.ds(base, chunk)],
                                v.at[pl.ds(0, chunk)])

        @pl.when(is_last)
        def _():
            for hbm, v in ((px0_hbm, px0_v), (py0_hbm, py0_v),
                           (px1_hbm, px1_v), (py1_hbm, py1_v)):
                pltpu.sync_copy(hbm.at[pl.ds(base, last)],
                                v.at[pl.ds(0, last)])

        big = jnp.full((L,), 1.0e30, jnp.float32)
        gx0_v[pl.ds(g, L)] = big
        gy0_v[pl.ds(g, L)] = big
        gx1_v[pl.ds(g, L)] = big
        gy1_v[pl.ds(g, L)] = big

        mw = jnp.zeros((L,), jnp.float32)
        for i in range(g // L):
            ordv = orig_v[pl.ds(i * L, L)]
            x0v = plsc.load_gather(rx0_v, [ordv])
            x1v = plsc.load_gather(rx1_v, [ordv])
            gx0_v[pl.ds(i * L, L)] = x0v
            gy0_v[pl.ds(i * L, L)] = plsc.load_gather(ry0_v, [ordv])
            gx1_v[pl.ds(i * L, L)] = x1v
            gy1_v[pl.ds(i * L, L)] = plsc.load_gather(ry1_v, [ordv])
            mw = jnp.maximum(mw, x1v - x0v)
        maxw = jnp.max(mw)

        zi = jnp.zeros((L,), jnp.int32)
        zf = jnp.zeros((L,), jnp.float32)
        c0 = zi

        def block(j, _):
            off = j * L
            px0 = px0_v[pl.ds(off, L)]
            py0 = py0_v[pl.ds(off, L)]
            px1 = px1_v[pl.ds(off, L)]
            py1 = py1_v[pl.ds(off, L)]
            parea = (px1 - px0) * (py1 - py0)

            tlo = px0 - maxw
            lo = zi
            hi = zi
            for s in steps:
                vlo = plsc.load_gather(gx0_v, [lo + (s - 1)])
                lo = jnp.where(vlo < tlo, lo + s, lo)
                vhi = plsc.load_gather(gx0_v, [hi + (s - 1)])
                hi = jnp.where(vhi <= px1, hi + s, hi)
            vlo = plsc.load_gather(gx0_v, [lo])
            lo = jnp.where(vlo < tlo, lo + 1, lo)
            vhi = plsc.load_gather(gx0_v, [hi])
            hi = jnp.where(vhi <= px1, hi + 1, hi)

            trip = jnp.max(hi - lo)
            send = jnp.full((L,), g, jnp.int32)

            def citer(k, carry):
                best, bidx = carry
                safe = jnp.minimum(lo + k, send)
                gx0 = plsc.load_gather(gx0_v, [safe])
                gy0 = plsc.load_gather(gy0_v, [safe])
                gx1 = plsc.load_gather(gx1_v, [safe])
                gy1 = plsc.load_gather(gy1_v, [safe])
                ga = (gx1 - gx0) * (gy1 - gy0)
                w = jnp.maximum(
                    jnp.minimum(px1, gx1) - jnp.maximum(px0, gx0), 0.0)
                h = jnp.maximum(
                    jnp.minimum(py1, gy1) - jnp.maximum(py0, gy0), 0.0)
                inter = w * h
                union = (parea + ga) - inter
                iou = inter / union
                upd = iou > best
                best = jnp.where(upd, iou, best)
                bidx = jnp.where(upd, safe, bidx)
                return best, bidx

            best, bidx = plsc.parallel_loop(
                0, trip, 1, unroll=4, carry=(zf, zi))(citer)

            below = best < FG_BG_THRESH
            orig = plsc.load_gather(orig_v, [bidx])
            labs = plsc.load_gather(gtl_v, [orig])
            cidx = jnp.where(below, zi, orig)
            labs = jnp.where(below, zi, labs)
            mv_v[pl.ds(off, L)] = best
            idx_v[pl.ds(off, L)] = cidx
            lab_v[pl.ds(off, L)] = labs
            return 0

        lax.fori_loop(0, jnp.where(is_last, nblk_last, nblk), block, 0)

        @pl.when(jnp.logical_not(is_last))
        def _():
            pltpu.sync_copy(lab_v.at[pl.ds(0, chunk)],
                            lab_hbm.at[pl.ds(base, chunk)])
            pltpu.sync_copy(mv_v.at[pl.ds(0, chunk)],
                            mv_hbm.at[pl.ds(base, chunk)])
            pltpu.sync_copy(idx_v.at[pl.ds(0, chunk)],
                            idx_hbm.at[pl.ds(base, chunk)])

        @pl.when(is_last)
        def _():
            pltpu.sync_copy(lab_v.at[pl.ds(0, last)],
                            lab_hbm.at[pl.ds(base, last)])
            pltpu.sync_copy(mv_v.at[pl.ds(0, last)],
                            mv_hbm.at[pl.ds(base, last)])
            pltpu.sync_copy(idx_v.at[pl.ds(0, last)],
                            idx_hbm.at[pl.ds(base, last)])

    return sc_call


def _make_tc_call(m, g):
    rows = m // 128
    grid = (rows // 8,)

    def body(px0_ref, py0_ref, px1_ref, py1_ref,
             gx0_ref, gy0_ref, gx1_ref, gy1_ref, gtl_ref,
             mv_ref, idx_ref, lab_ref):
        px0 = px0_ref[...]
        py0 = py0_ref[...]
        px1 = px1_ref[...]
        py1 = py1_ref[...]
        parea = (px1 - px0) * (py1 - py0)
        zf = jnp.zeros((8, 128), jnp.float32)
        zi = jnp.zeros((8, 128), jnp.int32)

        def giter(gi, carry):
            best, bidx, blab = carry
            gx0 = gx0_ref[gi]
            gy0 = gy0_ref[gi]
            gx1 = gx1_ref[gi]
            gy1 = gy1_ref[gi]
            garea = (gx1 - gx0) * (gy1 - gy0)
            w = jnp.maximum(
                jnp.minimum(px1, gx1) - jnp.maximum(px0, gx0), 0.0)
            h = jnp.maximum(
                jnp.minimum(py1, gy1) - jnp.maximum(py0, gy0), 0.0)
            inter = w * h
            union = (parea + garea) - inter
            iou = inter / union
            upd = iou > best
            best = jnp.where(upd, iou, best)
            bidx = jnp.where(upd, jnp.full((8, 128), gi, jnp.int32), bidx)
            blab = jnp.where(upd, jnp.full((8, 128), gtl_ref[gi],
                                           jnp.int32), blab)
            return best, bidx, blab

        best, bidx, blab = lax.fori_loop(0, g, giter, (zf, zi, zi),
                                         unroll=8)
        below = best < FG_BG_THRESH
        mv_ref[...] = best
        idx_ref[...] = jnp.where(below, 0, bidx)
        lab_ref[...] = jnp.where(below, 0, blab)

    blk = pl.BlockSpec((8, 128), lambda i: (i, 0))
    smem = pl.BlockSpec(memory_space=pltpu.SMEM)
    return pl.pallas_call(
        body,
        grid=grid,
        in_specs=[blk] * 4 + [smem] * 5,
        out_specs=[blk, blk, blk],
        out_shape=[
            jax.ShapeDtypeStruct((rows, 128), jnp.float32),
            jax.ShapeDtypeStruct((rows, 128), jnp.int32),
            jax.ShapeDtypeStruct((rows, 128), jnp.int32),
        ],
    )


def kernel(proposals, gt_boxes, gt_labels):
    n = proposals.shape[0]
    g = gt_boxes.shape[0]
    tc_m = min((n * 2 // 3) // 1024 * 1024, n - NW * L)
    s = n - tc_m
    order = jnp.argsort(gt_boxes[:, 0]).astype(jnp.int32)
    px0, py0 = proposals[:, 0], proposals[:, 1]
    px1, py1 = proposals[:, 2], proposals[:, 3]
    gx0, gy0 = gt_boxes[:, 0], gt_boxes[:, 1]
    gx1, gy1 = gt_boxes[:, 2], gt_boxes[:, 3]
    lab_sc, mv_sc, idx_sc = _make_sc_call(s, g)(
        px0[:s], py0[:s], px1[:s], py1[:s],
        gx0, gy0, gx1, gy1, order, gt_labels)
    rows = tc_m // 128
    mv_tc, idx_tc, lab_tc = _make_tc_call(tc_m, g)(
        px0[s:].reshape(rows, 128), py0[s:].reshape(rows, 128),
        px1[s:].reshape(rows, 128), py1[s:].reshape(rows, 128),
        gx0, gy0, gx1, gy1, gt_labels)
    mv_tc = mv_tc.reshape(-1)
    idx_tc = idx_tc.reshape(-1)
    lab_tc = lab_tc.reshape(-1)
    lab = jnp.concatenate([lab_sc, lab_tc])
    mv = jnp.concatenate([mv_sc, mv_tc])
    idx = jnp.concatenate([idx_sc, idx_tc])
    return lab, mv, idx
